# baseline (speedup 1.0000x reference)
import functools

import jax
import jax.numpy as jnp
from jax import lax
from jax.experimental import pallas as pl
from jax.experimental.pallas import tpu as pltpu
from jax.experimental.pallas import tpu_sc as plsc

B = 4096
L = 200
D = 64
NC = 2
NS = 16
NW = NC * NS
BPW = B // NW
C0 = 104
C1 = L - C0


def _pool_call(user_idxs, item_idxs, unt, int_, user_emb_W, item_emb_W):
  Bh = user_idxs.shape[0]
  BPW = Bh // NW
  mesh = plsc.VectorSubcoreMesh(core_axis_name="c", subcore_axis_name="s")
  out_t = (jax.ShapeDtypeStruct((Bh, D), jnp.float32),
           jax.ShapeDtypeStruct((Bh, D), jnp.float32))

  @functools.partial(
      pl.kernel, mesh=mesh, out_type=out_t,
      compiler_params=pltpu.CompilerParams(use_tc_tiling_on_sc=False),
      scratch_types=[
          pltpu.VMEM((BPW,), jnp.int32),
          pltpu.VMEM((BPW,), jnp.int32),
          pltpu.VMEM((BPW, L), jnp.int32),
          pltpu.VMEM((BPW, L), jnp.int32),
          pltpu.VMEM((L, D), jnp.float32),
          pltpu.VMEM((L, D), jnp.float32),
          pltpu.VMEM((BPW, D), jnp.float32),
          pltpu.VMEM((BPW, D), jnp.float32),
          pltpu.SemaphoreType.DMA,
          pltpu.SemaphoreType.DMA,
          pltpu.SemaphoreType.DMA,
      ])
  def pool(uidx_hbm, iidx_hbm, unt_hbm, int_hbm, uemb_hbm, iemb_hbm,
           uout_hbm, iout_hbm,
           uidx_v, iidx_v, uneigh_v, ineigh_v, buf_a, buf_b, uout_v, iout_v,
           sem_a, sem_b, sem_n):
    wid = lax.axis_index("s") * NC + lax.axis_index("c")
    base = wid * BPW
    pltpu.sync_copy(uidx_hbm.at[pl.ds(base, BPW)], uidx_v)
    pltpu.sync_copy(iidx_hbm.at[pl.ds(base, BPW)], iidx_v)
    pltpu.async_copy(unt_hbm.at[uidx_v], uneigh_v, sem_n)
    pltpu.async_copy(int_hbm.at[iidx_v], ineigh_v, sem_n)
    pltpu.make_async_copy(unt_hbm.at[pl.ds(0, BPW)], uneigh_v, sem_n).wait()
    pltpu.make_async_copy(int_hbm.at[pl.ds(0, BPW)], ineigh_v, sem_n).wait()

    def fire(emb_hbm, neigh_v, i, buf, sem):
      pltpu.async_copy(emb_hbm.at[neigh_v.at[i, pl.ds(0, C0)]],
                       buf.at[pl.ds(0, C0)], sem)
      pltpu.async_copy(emb_hbm.at[neigh_v.at[i, pl.ds(C0, C1)]],
                       buf.at[pl.ds(C0, C1)], sem)

    def drain(emb_hbm, buf, sem):
      pltpu.make_async_copy(emb_hbm.at[pl.ds(0, L)], buf, sem).wait()

    def reduce_into(buf, out_v, i):
      def body(r, accs):
        return tuple(accs[c] + buf[r, pl.ds(16 * c, 16)] for c in range(4))
      z = jnp.zeros((16,), jnp.float32)
      accs = lax.fori_loop(0, L, body, (z, z, z, z), unroll=8)
      for c in range(4):
        out_v[i, pl.ds(16 * c, 16)] = accs[c]

    def do_side(emb_hbm, neigh_v, out_v):
      fire(emb_hbm, neigh_v, 0, buf_a, sem_a)

      @pl.loop(0, BPW, step=2)
      def _(i):
        fire(emb_hbm, neigh_v, i + 1, buf_b, sem_b)
        drain(emb_hbm, buf_a, sem_a)
        reduce_into(buf_a, out_v, i)

        @pl.when(i + 2 < BPW)
        def _():
          fire(emb_hbm, neigh_v, i + 2, buf_a, sem_a)

        drain(emb_hbm, buf_b, sem_b)
        reduce_into(buf_b, out_v, i + 1)

    do_side(uemb_hbm, uneigh_v, uout_v)
    do_side(iemb_hbm, ineigh_v, iout_v)
    pltpu.sync_copy(uout_v, uout_hbm.at[pl.ds(base, BPW)])
    pltpu.sync_copy(iout_v, iout_hbm.at[pl.ds(base, BPW)])

  return pool(user_idxs, item_idxs, unt, int_, user_emb_W, item_emb_W)


def _mlp_call(u_sum, i_sum, w1, b1, w2, b2, w3, b3, w4, b4, w5, b5):
  Bh = u_sum.shape[0]
  n = Bh * D
  rows = 4096
  grid = n // rows
  elems = rows // D
  u3 = u_sum.reshape(grid, 1, rows)
  v3 = i_sum.reshape(grid, 1, rows)
  w1t = w1.T
  w2t = w2.T.astype(jnp.bfloat16)
  w3t = w3.T.astype(jnp.bfloat16)
  w4t = w4.T.astype(jnp.bfloat16)
  w5t = w5.T
  b1c = b1.reshape(-1, 1)
  b2c = b2.reshape(-1, 1)
  b3c = b3.reshape(-1, 1)
  b4c = b4.reshape(-1, 1)
  b5c = b5.reshape(-1, 1)
  s2 = jnp.kron(jnp.eye(elems, dtype=jnp.float32),
                jnp.full((D, 1), 1.0 / D, jnp.float32))

  def bdot(w_ref, x):
    return jnp.dot(w_ref[...], x.astype(jnp.bfloat16),
                   preferred_element_type=jnp.float32)

  def body(u_ref, v_ref, w1_ref, b1_ref, w2_ref, b2_ref, w3_ref, b3_ref,
           w4_ref, b4_ref, w5_ref, b5_ref, s_ref, o_ref):
    w1v = w1_ref[...]
    u = u_ref[...].reshape(1, rows)
    v = v_ref[...].reshape(1, rows)
    h = jnp.maximum(w1v[:, 0:1] * u + w1v[:, 1:2] * v + b1_ref[...], 0.0)
    h = jnp.maximum(bdot(w2_ref, h) + b2_ref[...], 0.0)
    h = jnp.maximum(bdot(w3_ref, h) + b3_ref[...], 0.0)
    h = jnp.maximum(bdot(w4_ref, h) + b4_ref[...], 0.0)
    o = jnp.dot(w5_ref[...], h, preferred_element_type=jnp.float32) + b5_ref[...]
    m = jnp.dot(o, s_ref[...], preferred_element_type=jnp.float32)
    o_ref[...] = jax.nn.sigmoid(m).reshape(1, 1, elems)

  def full(a):
    nd = a.ndim
    return pl.BlockSpec(a.shape, lambda g, _nd=nd: (0,) * _nd)

  out = pl.pallas_call(
      body,
      grid=(grid,),
      in_specs=[
          pl.BlockSpec((1, 1, rows), lambda g: (g, 0, 0)),
          pl.BlockSpec((1, 1, rows), lambda g: (g, 0, 0)),
          full(w1t), full(b1c), full(w2t), full(b2c), full(w3t), full(b3c),
          full(w4t), full(b4c), full(w5t), full(b5c), full(s2),
      ],
      out_specs=pl.BlockSpec((1, 1, elems), lambda g: (g, 0, 0)),
      out_shape=jax.ShapeDtypeStruct((grid, 1, elems), jnp.float32),
  )(u3, v3, w1t, b1c, w2t, b2c, w3t, b3c, w4t, b4c, w5t, b5c, s2)
  return out.reshape(Bh)


def kernel(user_idxs, item_idxs, user_idx_tensor, item_idx_tensor,
           user_emb_W, item_emb_W, w1, b1, w2, b2, w3, b3, w4, b4, w5, b5):
  z32 = lax.shift_right_logical(user_idxs[0], 31).astype(jnp.int32)
  untz = user_idx_tensor.astype(jnp.int32) + z32
  intz = item_idx_tensor.astype(jnp.int32) + z32
  u_sum, i_sum = _pool_call(user_idxs.astype(jnp.int32),
                            item_idxs.astype(jnp.int32),
                            untz, intz, user_emb_W, item_emb_W)
  return _mlp_call(u_sum, i_sum, w1, b1, w2, b2, w3, b3, w4, b4, w5, b5)


# speedup vs baseline: 2.2059x; 2.2059x over previous
import functools

import jax
import jax.numpy as jnp
from jax import lax
from jax.experimental import pallas as pl
from jax.experimental.pallas import tpu as pltpu
from jax.experimental.pallas import tpu_sc as plsc

B = 4096
L = 200
D = 64
NC = 2
NS = 16
NW = NC * NS
BPW = B // NW
C0 = 104
C1 = L - C0


def _pool_call(uneigh, ineigh, user_emb_W, item_emb_W):
  Bh = uneigh.shape[0]
  BPW = Bh // NW
  mesh = plsc.VectorSubcoreMesh(core_axis_name="c", subcore_axis_name="s")
  out_t = (jax.ShapeDtypeStruct((Bh, D), jnp.float32),
           jax.ShapeDtypeStruct((Bh, D), jnp.float32))

  @functools.partial(
      pl.kernel, mesh=mesh, out_type=out_t,
      compiler_params=pltpu.CompilerParams(use_tc_tiling_on_sc=False),
      scratch_types=[
          pltpu.VMEM((BPW, L), jnp.int32),
          pltpu.VMEM((BPW, L), jnp.int32),
          pltpu.VMEM((L, D), jnp.float32),
          pltpu.VMEM((L, D), jnp.float32),
          pltpu.VMEM((BPW, D), jnp.float32),
          pltpu.VMEM((BPW, D), jnp.float32),
          pltpu.SemaphoreType.DMA,
          pltpu.SemaphoreType.DMA,
          pltpu.SemaphoreType.DMA,
      ])
  def pool(un_hbm, in_hbm, uemb_hbm, iemb_hbm,
           uout_hbm, iout_hbm,
           uneigh_v, ineigh_v, buf_a, buf_b, uout_v, iout_v,
           sem_a, sem_b, sem_n):
    wid = lax.axis_index("s") * NC + lax.axis_index("c")
    base = wid * BPW
    pltpu.async_copy(un_hbm.at[pl.ds(base, BPW)], uneigh_v, sem_n)
    pltpu.async_copy(in_hbm.at[pl.ds(base, BPW)], ineigh_v, sem_n)
    pltpu.make_async_copy(un_hbm.at[pl.ds(0, BPW)], uneigh_v, sem_n).wait()
    pltpu.make_async_copy(in_hbm.at[pl.ds(0, BPW)], ineigh_v, sem_n).wait()

    def fire(emb_hbm, neigh_v, i, buf, sem):
      pltpu.async_copy(emb_hbm.at[neigh_v.at[i, pl.ds(0, C0)]],
                       buf.at[pl.ds(0, C0)], sem)
      pltpu.async_copy(emb_hbm.at[neigh_v.at[i, pl.ds(C0, C1)]],
                       buf.at[pl.ds(C0, C1)], sem)

    def drain(emb_hbm, buf, sem):
      pltpu.make_async_copy(emb_hbm.at[pl.ds(0, L)], buf, sem).wait()

    def reduce_into(buf, out_v, i):
      def body(r, accs):
        return tuple(accs[c] + buf[r, pl.ds(16 * c, 16)] for c in range(4))
      z = jnp.zeros((16,), jnp.float32)
      accs = lax.fori_loop(0, L, body, (z, z, z, z), unroll=8)
      for c in range(4):
        out_v[i, pl.ds(16 * c, 16)] = accs[c]

    def do_side(emb_hbm, neigh_v, out_v):
      fire(emb_hbm, neigh_v, 0, buf_a, sem_a)

      @pl.loop(0, BPW, step=2)
      def _(i):
        fire(emb_hbm, neigh_v, i + 1, buf_b, sem_b)
        drain(emb_hbm, buf_a, sem_a)
        reduce_into(buf_a, out_v, i)

        @pl.when(i + 2 < BPW)
        def _():
          fire(emb_hbm, neigh_v, i + 2, buf_a, sem_a)

        drain(emb_hbm, buf_b, sem_b)
        reduce_into(buf_b, out_v, i + 1)

    do_side(uemb_hbm, uneigh_v, uout_v)
    do_side(iemb_hbm, ineigh_v, iout_v)
    pltpu.sync_copy(uout_v, uout_hbm.at[pl.ds(base, BPW)])
    pltpu.sync_copy(iout_v, iout_hbm.at[pl.ds(base, BPW)])

  return pool(uneigh, ineigh, user_emb_W, item_emb_W)


def _mlp_call(u_sum, i_sum, w1, b1, w2, b2, w3, b3, w4, b4, w5, b5):
  Bh = u_sum.shape[0]
  n = Bh * D
  rows = 4096
  grid = n // rows
  elems = rows // D
  u3 = u_sum.reshape(grid, 1, rows)
  v3 = i_sum.reshape(grid, 1, rows)
  w1t = w1.T
  w2t = w2.T.astype(jnp.bfloat16)
  w3t = w3.T.astype(jnp.bfloat16)
  w4t = w4.T.astype(jnp.bfloat16)
  w5t = w5.T
  b1c = b1.reshape(-1, 1)
  b2c = b2.reshape(-1, 1)
  b3c = b3.reshape(-1, 1)
  b4c = b4.reshape(-1, 1)
  b5c = b5.reshape(-1, 1)
  s2 = jnp.kron(jnp.eye(elems, dtype=jnp.float32),
                jnp.full((D, 1), 1.0 / D, jnp.float32))

  def bdot(w_ref, x):
    return jnp.dot(w_ref[...], x.astype(jnp.bfloat16),
                   preferred_element_type=jnp.float32)

  def body(u_ref, v_ref, w1_ref, b1_ref, w2_ref, b2_ref, w3_ref, b3_ref,
           w4_ref, b4_ref, w5_ref, b5_ref, s_ref, o_ref):
    w1v = w1_ref[...]
    u = u_ref[...].reshape(1, rows)
    v = v_ref[...].reshape(1, rows)
    h = jnp.maximum(w1v[:, 0:1] * u + w1v[:, 1:2] * v + b1_ref[...], 0.0)
    h = jnp.maximum(bdot(w2_ref, h) + b2_ref[...], 0.0)
    h = jnp.maximum(bdot(w3_ref, h) + b3_ref[...], 0.0)
    h = jnp.maximum(bdot(w4_ref, h) + b4_ref[...], 0.0)
    o = jnp.dot(w5_ref[...], h, preferred_element_type=jnp.float32) + b5_ref[...]
    m = jnp.dot(o, s_ref[...], preferred_element_type=jnp.float32)
    o_ref[...] = jax.nn.sigmoid(m).reshape(1, 1, elems)

  def full(a):
    nd = a.ndim
    return pl.BlockSpec(a.shape, lambda g, _nd=nd: (0,) * _nd)

  out = pl.pallas_call(
      body,
      grid=(grid,),
      in_specs=[
          pl.BlockSpec((1, 1, rows), lambda g: (g, 0, 0)),
          pl.BlockSpec((1, 1, rows), lambda g: (g, 0, 0)),
          full(w1t), full(b1c), full(w2t), full(b2c), full(w3t), full(b3c),
          full(w4t), full(b4c), full(w5t), full(b5c), full(s2),
      ],
      out_specs=pl.BlockSpec((1, 1, elems), lambda g: (g, 0, 0)),
      out_shape=jax.ShapeDtypeStruct((grid, 1, elems), jnp.float32),
  )(u3, v3, w1t, b1c, w2t, b2c, w3t, b3c, w4t, b4c, w5t, b5c, s2)
  return out.reshape(Bh)


def kernel(user_idxs, item_idxs, user_idx_tensor, item_idx_tensor,
           user_emb_W, item_emb_W, w1, b1, w2, b2, w3, b3, w4, b4, w5, b5):
  z32 = lax.shift_right_logical(user_idxs[0], 31).astype(jnp.int32)
  uneigh = jnp.take(user_idx_tensor + z32, user_idxs, axis=0).astype(jnp.int32)
  ineigh = jnp.take(item_idx_tensor + z32, item_idxs, axis=0).astype(jnp.int32)
  u_sum, i_sum = _pool_call(uneigh, ineigh, user_emb_W, item_emb_W)
  return _mlp_call(u_sum, i_sum, w1, b1, w2, b2, w3, b3, w4, b4, w5, b5)


# speedup vs baseline: 2.4658x; 1.1178x over previous
import functools

import jax
import jax.numpy as jnp
from jax import lax
from jax.experimental import pallas as pl
from jax.experimental.pallas import tpu as pltpu
from jax.experimental.pallas import tpu_sc as plsc

B = 4096
L = 200
D = 64
NC = 2
NS = 16
NW = NC * NS
BPW = B // NW
C0 = 104
C1 = L - C0


def _pool_call(uneigh, ineigh, user_emb_W, item_emb_W):
  Bh = uneigh.shape[0]
  BPW = Bh // NW
  mesh = plsc.VectorSubcoreMesh(core_axis_name="c", subcore_axis_name="s")
  out_t = (jax.ShapeDtypeStruct((Bh, D), jnp.float32),
           jax.ShapeDtypeStruct((Bh, D), jnp.float32))

  @functools.partial(
      pl.kernel, mesh=mesh, out_type=out_t,
      compiler_params=pltpu.CompilerParams(use_tc_tiling_on_sc=False),
      scratch_types=[
          pltpu.VMEM((BPW, L), jnp.int32),
          pltpu.VMEM((BPW, L), jnp.int32),
          pltpu.VMEM((L, D), jnp.float32),
          pltpu.VMEM((L, D), jnp.float32),
          pltpu.VMEM((L, D), jnp.float32),
          pltpu.VMEM((L, D), jnp.float32),
          pltpu.VMEM((BPW, D), jnp.float32),
          pltpu.VMEM((BPW, D), jnp.float32),
          pltpu.SemaphoreType.DMA,
          pltpu.SemaphoreType.DMA,
          pltpu.SemaphoreType.DMA,
          pltpu.SemaphoreType.DMA,
          pltpu.SemaphoreType.DMA,
      ])
  def pool(un_hbm, in_hbm, uemb_hbm, iemb_hbm,
           uout_hbm, iout_hbm,
           uneigh_v, ineigh_v, buf_a, buf_b, buf_c, buf_d, uout_v, iout_v,
           sem_a, sem_b, sem_c, sem_d, sem_n):
    wid = lax.axis_index("s") * NC + lax.axis_index("c")
    base = wid * BPW
    pltpu.async_copy(un_hbm.at[pl.ds(base, BPW)], uneigh_v, sem_n)
    pltpu.async_copy(in_hbm.at[pl.ds(base, BPW)], ineigh_v, sem_n)
    pltpu.make_async_copy(un_hbm.at[pl.ds(0, BPW)], uneigh_v, sem_n).wait()
    pltpu.make_async_copy(in_hbm.at[pl.ds(0, BPW)], ineigh_v, sem_n).wait()

    def fire(emb_hbm, neigh_v, i, buf, sem):
      pltpu.async_copy(emb_hbm.at[neigh_v.at[i, pl.ds(0, C0)]],
                       buf.at[pl.ds(0, C0)], sem)
      pltpu.async_copy(emb_hbm.at[neigh_v.at[i, pl.ds(C0, C1)]],
                       buf.at[pl.ds(C0, C1)], sem)

    def drain(emb_hbm, buf, sem):
      pltpu.make_async_copy(emb_hbm.at[pl.ds(0, L)], buf, sem).wait()

    def reduce_into(buf, out_v, i):
      def body(r, accs):
        return tuple(accs[c] + buf[r, pl.ds(16 * c, 16)] for c in range(4))
      z = jnp.zeros((16,), jnp.float32)
      accs = lax.fori_loop(0, L, body, (z, z, z, z), unroll=8)
      for c in range(4):
        out_v[i, pl.ds(16 * c, 16)] = accs[c]

    bufs = ((buf_a, sem_a), (buf_b, sem_b), (buf_c, sem_c), (buf_d, sem_d))

    def do_side(emb_hbm, neigh_v, out_v):
      for k, (buf, sem) in enumerate(bufs):
        fire(emb_hbm, neigh_v, k, buf, sem)

      @pl.loop(0, BPW, step=4)
      def _(i):
        for k, (buf, sem) in enumerate(bufs):
          drain(emb_hbm, buf, sem)
          reduce_into(buf, out_v, i + k)

          @pl.when(i + 4 + k < BPW)
          def _():
            fire(emb_hbm, neigh_v, i + 4 + k, buf, sem)

    do_side(uemb_hbm, uneigh_v, uout_v)
    do_side(iemb_hbm, ineigh_v, iout_v)
    pltpu.sync_copy(uout_v, uout_hbm.at[pl.ds(base, BPW)])
    pltpu.sync_copy(iout_v, iout_hbm.at[pl.ds(base, BPW)])

  return pool(uneigh, ineigh, user_emb_W, item_emb_W)


def _mlp_call(u_sum, i_sum, w1, b1, w2, b2, w3, b3, w4, b4, w5, b5):
  Bh = u_sum.shape[0]
  n = Bh * D
  rows = 4096
  grid = n // rows
  elems = rows // D
  u3 = u_sum.reshape(grid, 1, rows)
  v3 = i_sum.reshape(grid, 1, rows)
  w1t = w1.T
  w2t = w2.T.astype(jnp.bfloat16)
  w3t = w3.T.astype(jnp.bfloat16)
  w4t = w4.T.astype(jnp.bfloat16)
  w5t = w5.T
  b1c = b1.reshape(-1, 1)
  b2c = b2.reshape(-1, 1)
  b3c = b3.reshape(-1, 1)
  b4c = b4.reshape(-1, 1)
  b5c = b5.reshape(-1, 1)
  s2 = jnp.kron(jnp.eye(elems, dtype=jnp.float32),
                jnp.full((D, 1), 1.0 / D, jnp.float32))

  def bdot(w_ref, x):
    return jnp.dot(w_ref[...], x.astype(jnp.bfloat16),
                   preferred_element_type=jnp.float32)

  def body(u_ref, v_ref, w1_ref, b1_ref, w2_ref, b2_ref, w3_ref, b3_ref,
           w4_ref, b4_ref, w5_ref, b5_ref, s_ref, o_ref):
    w1v = w1_ref[...]
    u = u_ref[...].reshape(1, rows)
    v = v_ref[...].reshape(1, rows)
    h = jnp.maximum(w1v[:, 0:1] * u + w1v[:, 1:2] * v + b1_ref[...], 0.0)
    h = jnp.maximum(bdot(w2_ref, h) + b2_ref[...], 0.0)
    h = jnp.maximum(bdot(w3_ref, h) + b3_ref[...], 0.0)
    h = jnp.maximum(bdot(w4_ref, h) + b4_ref[...], 0.0)
    o = jnp.dot(w5_ref[...], h, preferred_element_type=jnp.float32) + b5_ref[...]
    m = jnp.dot(o, s_ref[...], preferred_element_type=jnp.float32)
    o_ref[...] = jax.nn.sigmoid(m).reshape(1, 1, elems)

  def full(a):
    nd = a.ndim
    return pl.BlockSpec(a.shape, lambda g, _nd=nd: (0,) * _nd)

  out = pl.pallas_call(
      body,
      grid=(grid,),
      in_specs=[
          pl.BlockSpec((1, 1, rows), lambda g: (g, 0, 0)),
          pl.BlockSpec((1, 1, rows), lambda g: (g, 0, 0)),
          full(w1t), full(b1c), full(w2t), full(b2c), full(w3t), full(b3c),
          full(w4t), full(b4c), full(w5t), full(b5c), full(s2),
      ],
      out_specs=pl.BlockSpec((1, 1, elems), lambda g: (g, 0, 0)),
      out_shape=jax.ShapeDtypeStruct((grid, 1, elems), jnp.float32),
  )(u3, v3, w1t, b1c, w2t, b2c, w3t, b3c, w4t, b4c, w5t, b5c, s2)
  return out.reshape(Bh)


def kernel(user_idxs, item_idxs, user_idx_tensor, item_idx_tensor,
           user_emb_W, item_emb_W, w1, b1, w2, b2, w3, b3, w4, b4, w5, b5):
  z32 = lax.shift_right_logical(user_idxs[0], 31).astype(jnp.int32)
  uneigh = jnp.take(user_idx_tensor + z32, user_idxs, axis=0).astype(jnp.int32)
  ineigh = jnp.take(item_idx_tensor + z32, item_idxs, axis=0).astype(jnp.int32)
  u_sum, i_sum = _pool_call(uneigh, ineigh, user_emb_W, item_emb_W)
  return _mlp_call(u_sum, i_sum, w1, b1, w2, b2, w3, b3, w4, b4, w5, b5)
